# Initial kernel scaffold; baseline (speedup 1.0000x reference)
import jax
import jax.numpy as jnp
from jax import lax
from jax.experimental import pallas as pl
from jax.experimental.pallas import tpu as pltpu
from jax.experimental.pallas import tpu_sc as plsc

N_NODES = 10000
E_EDGES = 320000
FEAT = 128
NCOLS = 8
NEG_SLOPE = 0.2

NC = 2
NS = 16
NTILES = NC * NS
CHUNK = E_EDGES // NTILES
LANES = 16
VECS = CHUNK // LANES
N_PAD = 10240


def _project_body(x_ref, w_ref, c_ref, out_ref):
    g = jnp.dot(w_ref[...], c_ref[...], preferred_element_type=jnp.float32)
    out_ref[...] = lax.dot_general(
        g, x_ref[...], (((0,), (1,)), ((), ())),
        preferred_element_type=jnp.float32)


def _project(xp, W, C):
    grid = 10
    bn = N_PAD // grid
    return pl.pallas_call(
        _project_body,
        grid=(grid,),
        in_specs=[
            pl.BlockSpec((bn, FEAT), lambda i: (i, 0)),
            pl.BlockSpec((FEAT, FEAT), lambda i: (0, 0)),
            pl.BlockSpec((FEAT, NCOLS), lambda i: (0, 0)),
        ],
        out_specs=pl.BlockSpec((NCOLS, bn), lambda i: (0, i)),
        out_shape=jax.ShapeDtypeStruct((NCOLS, N_PAD), jnp.float32),
    )(xp, W, C)


def _edge_body(pt_hbm, src_hbm, dst_hbm, ew_hbm, parts_hbm,
               asrc_v, adst_v, z_v, srcs_v, dsts_v, ews_v, accn_v, accd_v):
    c = lax.axis_index("c")
    s = lax.axis_index("s")
    tid = s * NC + c
    base = tid * CHUNK

    pltpu.sync_copy(pt_hbm.at[0], asrc_v)
    pltpu.sync_copy(pt_hbm.at[1], adst_v)
    pltpu.sync_copy(pt_hbm.at[2], z_v)
    pltpu.sync_copy(src_hbm.at[pl.ds(base, CHUNK)], srcs_v)
    pltpu.sync_copy(dst_hbm.at[pl.ds(base, CHUNK)], dsts_v)
    pltpu.sync_copy(ew_hbm.at[pl.ds(base, CHUNK)], ews_v)

    zeros = jnp.zeros((LANES,), jnp.float32)

    def zero_body(i, _):
        sl = pl.ds(i * LANES, LANES)
        accn_v[sl] = zeros
        accd_v[sl] = zeros
        return 0

    lax.fori_loop(0, N_PAD // LANES, zero_body, 0)

    def edge_body(i, _):
        sl = pl.ds(i * LANES, LANES)
        sidx = srcs_v[sl]
        didx = dsts_v[sl]
        ew = ews_v[sl]
        ga = plsc.load_gather(asrc_v, [sidx])
        gb = plsc.load_gather(adst_v, [didx])
        gz = plsc.load_gather(z_v, [sidx])
        a = ga + gb
        a = jnp.where(a >= 0.0, a, a * NEG_SLOPE)
        ex = jnp.exp(a)
        plsc.addupdate_scatter(accd_v, [didx], ex)
        plsc.addupdate_scatter(accn_v, [didx], ex * ew * gz)
        return 0

    lax.fori_loop(0, VECS, edge_body, 0)

    pltpu.sync_copy(accn_v, parts_hbm.at[tid, 0])
    pltpu.sync_copy(accd_v, parts_hbm.at[tid, 1])


def _edge_phase(pt, src, dst, ew):
    fn = pl.kernel(
        _edge_body,
        out_type=jax.ShapeDtypeStruct((NTILES, 2, N_PAD), jnp.float32),
        mesh=plsc.VectorSubcoreMesh(core_axis_name="c", subcore_axis_name="s"),
        scratch_types=[
            pltpu.VMEM((N_PAD,), jnp.float32),
            pltpu.VMEM((N_PAD,), jnp.float32),
            pltpu.VMEM((N_PAD,), jnp.float32),
            pltpu.VMEM((CHUNK,), jnp.int32),
            pltpu.VMEM((CHUNK,), jnp.int32),
            pltpu.VMEM((CHUNK,), jnp.float32),
            pltpu.VMEM((N_PAD,), jnp.float32),
            pltpu.VMEM((N_PAD,), jnp.float32),
        ],
    )
    return fn(pt, src, dst, ew)


def _finish_body(parts_ref, bias_ref, fcw_ref, fcb_ref, out_ref):
    num = jnp.sum(parts_ref[:, 0, :, :], axis=0)
    den = jnp.sum(parts_ref[:, 1, :, :], axis=0)
    const = jnp.sum(bias_ref[...] * fcw_ref[...]) + fcb_ref[0, 0]
    out_ref[...] = num / (den + 1e-16) + const


def _finish(parts, bias, fc_W, fc_b):
    return pl.pallas_call(
        _finish_body,
        out_shape=jax.ShapeDtypeStruct((N_PAD // FEAT, FEAT), jnp.float32),
    )(parts.reshape(NTILES, 2, N_PAD // FEAT, FEAT),
      bias.reshape(1, FEAT), fc_W.reshape(1, FEAT), fc_b.reshape(1, 1))


@jax.jit
def kernel(x, edge_index, edge_weight, W, att_src, att_dst, bias, fc_W, fc_b):
    xp = jnp.zeros((N_PAD, FEAT), jnp.float32).at[:N_NODES].set(x)
    C = jnp.concatenate(
        [att_src[0][:, None], att_dst[0][:, None], fc_W,
         jnp.zeros((FEAT, NCOLS - 3), jnp.float32)], axis=1)
    pt = _project(xp, W, C)
    parts = _edge_phase(pt, edge_index[0], edge_index[1], edge_weight)
    out = _finish(parts, bias, fc_W, fc_b)
    return out.reshape(-1)[:N_NODES].reshape(N_NODES, 1)


# speedup vs baseline: 133.4007x; 133.4007x over previous
import jax
import jax.numpy as jnp
from jax import lax
from jax.experimental import pallas as pl
from jax.experimental.pallas import tpu as pltpu
from jax.experimental.pallas import tpu_sc as plsc

N_NODES = 10000
E_EDGES = 320000
FEAT = 128
NCOLS = 8
NEG_SLOPE = 0.2

NC = 2
NS = 16
NTILES = NC * NS
CHUNK = E_EDGES // NTILES
LANES = 16
VECS = CHUNK // LANES
N_PAD = 10240


def _project_body(x_ref, w_ref, c_ref, out_ref):
    g = jnp.dot(w_ref[...], c_ref[...], preferred_element_type=jnp.float32)
    out_ref[...] = lax.dot_general(
        g, x_ref[...], (((0,), (1,)), ((), ())),
        preferred_element_type=jnp.float32)


def _project(xp, W, C):
    grid = 10
    bn = N_PAD // grid
    return pl.pallas_call(
        _project_body,
        grid=(grid,),
        in_specs=[
            pl.BlockSpec((bn, FEAT), lambda i: (i, 0)),
            pl.BlockSpec((FEAT, FEAT), lambda i: (0, 0)),
            pl.BlockSpec((FEAT, NCOLS), lambda i: (0, 0)),
        ],
        out_specs=pl.BlockSpec((NCOLS, bn), lambda i: (0, i)),
        out_shape=jax.ShapeDtypeStruct((NCOLS, N_PAD), jnp.float32),
    )(xp, W, C)


def _edge_body(pt_hbm, src_hbm, dst_hbm, ew_hbm, parts_hbm,
               asrc_v, adst_v, z_v, srcs_v, dsts_v, ews_v, accn_v, accd_v):
    c = lax.axis_index("c")
    s = lax.axis_index("s")
    tid = s * NC + c
    base = tid * CHUNK

    pltpu.sync_copy(pt_hbm.at[0], asrc_v)
    pltpu.sync_copy(pt_hbm.at[1], adst_v)
    pltpu.sync_copy(pt_hbm.at[2], z_v)
    pltpu.sync_copy(src_hbm.at[pl.ds(base, CHUNK)], srcs_v)
    pltpu.sync_copy(dst_hbm.at[pl.ds(base, CHUNK)], dsts_v)
    pltpu.sync_copy(ew_hbm.at[pl.ds(base, CHUNK)], ews_v)

    zeros = jnp.zeros((LANES,), jnp.float32)

    def zero_body(i, _):
        sl = pl.ds(i * LANES, LANES)
        accn_v[sl] = zeros
        accd_v[sl] = zeros
        return 0

    lax.fori_loop(0, N_PAD // LANES, zero_body, 0)

    def edge_body(i, _):
        sl = pl.ds(i * LANES, LANES)
        sidx = srcs_v[sl]
        didx = dsts_v[sl]
        ew = ews_v[sl]
        ga = plsc.load_gather(asrc_v, [sidx])
        gb = plsc.load_gather(adst_v, [didx])
        gz = plsc.load_gather(z_v, [sidx])
        a = ga + gb
        a = jnp.where(a >= 0.0, a, a * NEG_SLOPE)
        ex = jnp.exp(a)
        plsc.addupdate_scatter(accd_v, [didx], ex)
        plsc.addupdate_scatter(accn_v, [didx], ex * ew * gz)
        return 0

    lax.fori_loop(0, VECS, edge_body, 0)

    pltpu.sync_copy(accn_v, parts_hbm.at[tid, 0])
    pltpu.sync_copy(accd_v, parts_hbm.at[tid, 1])


def _edge_phase(pt, src, dst, ew):
    fn = pl.kernel(
        _edge_body,
        out_type=jax.ShapeDtypeStruct((NTILES, 2, N_PAD), jnp.float32),
        mesh=plsc.VectorSubcoreMesh(core_axis_name="c", subcore_axis_name="s",
                                    num_cores=NC, num_subcores=NS),
        scratch_types=[
            pltpu.VMEM((N_PAD,), jnp.float32),
            pltpu.VMEM((N_PAD,), jnp.float32),
            pltpu.VMEM((N_PAD,), jnp.float32),
            pltpu.VMEM((CHUNK,), jnp.int32),
            pltpu.VMEM((CHUNK,), jnp.int32),
            pltpu.VMEM((CHUNK,), jnp.float32),
            pltpu.VMEM((N_PAD,), jnp.float32),
            pltpu.VMEM((N_PAD,), jnp.float32),
        ],
        compiler_params=pltpu.CompilerParams(needs_layout_passes=False),
    )
    return fn(pt, src, dst, ew)


def _finish_body(parts_ref, bias_ref, fcw_ref, fcb_ref, out_ref):
    num = jnp.sum(parts_ref[:, 0, :, :], axis=0)
    den = jnp.sum(parts_ref[:, 1, :, :], axis=0)
    const = jnp.sum(bias_ref[...] * fcw_ref[...]) + fcb_ref[0, 0]
    out_ref[...] = num / (den + 1e-16) + const


def _finish(parts, bias, fc_W, fc_b):
    return pl.pallas_call(
        _finish_body,
        out_shape=jax.ShapeDtypeStruct((N_PAD // FEAT, FEAT), jnp.float32),
    )(parts.reshape(NTILES, 2, N_PAD // FEAT, FEAT),
      bias.reshape(1, FEAT), fc_W.reshape(1, FEAT), fc_b.reshape(1, 1))


@jax.jit
def kernel(x, edge_index, edge_weight, W, att_src, att_dst, bias, fc_W, fc_b):
    xp = jnp.zeros((N_PAD, FEAT), jnp.float32).at[:N_NODES].set(x)
    C = jnp.concatenate(
        [att_src[0][:, None], att_dst[0][:, None], fc_W,
         jnp.zeros((FEAT, NCOLS - 3), jnp.float32)], axis=1)
    pt = _project(xp, W, C)
    parts = _edge_phase(pt, edge_index[0], edge_index[1], edge_weight)
    out = _finish(parts, bias, fc_W, fc_b)
    return out.reshape(-1)[:N_NODES].reshape(N_NODES, 1)


# speedup vs baseline: 154.3288x; 1.1569x over previous
import jax
import jax.numpy as jnp
from jax import lax
from jax.experimental import pallas as pl
from jax.experimental.pallas import tpu as pltpu
from jax.experimental.pallas import tpu_sc as plsc

N_NODES = 10000
E_EDGES = 320000
FEAT = 128
NCOLS = 8
NEG_SLOPE = 0.2

NC = 2
NS = 16
NTILES = NC * NS
CHUNK = E_EDGES // NTILES
LANES = 16
VECS = CHUNK // LANES
N_PAD = 10240


def _project_body(x_ref, w_ref, c_ref, out_ref):
    g = jnp.dot(w_ref[...], c_ref[...], preferred_element_type=jnp.float32)
    out_ref[...] = lax.dot_general(
        g, x_ref[...], (((0,), (1,)), ((), ())),
        preferred_element_type=jnp.float32)


def _project(xp, W, C):
    grid = 10
    bn = N_PAD // grid
    return pl.pallas_call(
        _project_body,
        grid=(grid,),
        in_specs=[
            pl.BlockSpec((bn, FEAT), lambda i: (i, 0)),
            pl.BlockSpec((FEAT, FEAT), lambda i: (0, 0)),
            pl.BlockSpec((FEAT, NCOLS), lambda i: (0, 0)),
        ],
        out_specs=pl.BlockSpec((NCOLS, bn), lambda i: (0, i)),
        out_shape=jax.ShapeDtypeStruct((NCOLS, N_PAD), jnp.float32),
    )(xp, W, C)


def _edge_body(pt_hbm, src_hbm, dst_hbm, ew_hbm, parts_hbm,
               asrc_v, adst_v, z_v, srcs_v, dsts_v, ews_v, accn_v, accd_v):
    c = lax.axis_index("c")
    s = lax.axis_index("s")
    tid = s * NC + c
    base = tid * CHUNK

    pltpu.sync_copy(pt_hbm.at[0], asrc_v)
    pltpu.sync_copy(pt_hbm.at[1], adst_v)
    pltpu.sync_copy(pt_hbm.at[2], z_v)
    pltpu.sync_copy(src_hbm.at[pl.ds(base, CHUNK)], srcs_v)
    pltpu.sync_copy(dst_hbm.at[pl.ds(base, CHUNK)], dsts_v)
    pltpu.sync_copy(ew_hbm.at[pl.ds(base, CHUNK)], ews_v)

    zeros = jnp.zeros((LANES,), jnp.float32)

    @plsc.parallel_loop(0, N_PAD, step=LANES, unroll=8)
    def _zero(i):
        sl = pl.ds(i, LANES)
        accn_v[sl] = zeros
        accd_v[sl] = zeros

    @plsc.parallel_loop(0, CHUNK, step=LANES, unroll=5)
    def _edges(i):
        sl = pl.ds(i, LANES)
        sidx = srcs_v[sl]
        didx = dsts_v[sl]
        ew = ews_v[sl]
        ga = plsc.load_gather(asrc_v, [sidx])
        gb = plsc.load_gather(adst_v, [didx])
        gz = plsc.load_gather(z_v, [sidx])
        a = ga + gb
        a = jnp.where(a >= 0.0, a, a * NEG_SLOPE)
        ex = jnp.exp(a)
        plsc.addupdate_scatter(accd_v, [didx], ex)
        plsc.addupdate_scatter(accn_v, [didx], ex * ew * gz)

    pltpu.sync_copy(accn_v, parts_hbm.at[tid, 0])
    pltpu.sync_copy(accd_v, parts_hbm.at[tid, 1])


def _edge_phase(pt, src, dst, ew):
    fn = pl.kernel(
        _edge_body,
        out_type=jax.ShapeDtypeStruct((NTILES, 2, N_PAD), jnp.float32),
        mesh=plsc.VectorSubcoreMesh(core_axis_name="c", subcore_axis_name="s",
                                    num_cores=NC, num_subcores=NS),
        scratch_types=[
            pltpu.VMEM((N_PAD,), jnp.float32),
            pltpu.VMEM((N_PAD,), jnp.float32),
            pltpu.VMEM((N_PAD,), jnp.float32),
            pltpu.VMEM((CHUNK,), jnp.int32),
            pltpu.VMEM((CHUNK,), jnp.int32),
            pltpu.VMEM((CHUNK,), jnp.float32),
            pltpu.VMEM((N_PAD,), jnp.float32),
            pltpu.VMEM((N_PAD,), jnp.float32),
        ],
        compiler_params=pltpu.CompilerParams(needs_layout_passes=False),
    )
    return fn(pt, src, dst, ew)


def _finish_body(parts_ref, bias_ref, fcw_ref, fcb_ref, out_ref):
    num = jnp.sum(parts_ref[:, 0, :, :], axis=0)
    den = jnp.sum(parts_ref[:, 1, :, :], axis=0)
    const = jnp.sum(bias_ref[...] * fcw_ref[...]) + fcb_ref[0, 0]
    out_ref[...] = num / (den + 1e-16) + const


def _finish(parts, bias, fc_W, fc_b):
    return pl.pallas_call(
        _finish_body,
        out_shape=jax.ShapeDtypeStruct((N_PAD // FEAT, FEAT), jnp.float32),
    )(parts.reshape(NTILES, 2, N_PAD // FEAT, FEAT),
      bias.reshape(1, FEAT), fc_W.reshape(1, FEAT), fc_b.reshape(1, 1))


@jax.jit
def kernel(x, edge_index, edge_weight, W, att_src, att_dst, bias, fc_W, fc_b):
    xp = jnp.zeros((N_PAD, FEAT), jnp.float32).at[:N_NODES].set(x)
    C = jnp.concatenate(
        [att_src[0][:, None], att_dst[0][:, None], fc_W,
         jnp.zeros((FEAT, NCOLS - 3), jnp.float32)], axis=1)
    pt = _project(xp, W, C)
    parts = _edge_phase(pt, edge_index[0], edge_index[1], edge_weight)
    out = _finish(parts, bias, fc_W, fc_b)
    return out.reshape(-1)[:N_NODES].reshape(N_NODES, 1)


# speedup vs baseline: 188.9880x; 1.2246x over previous
import jax
import jax.numpy as jnp
from jax import lax
from jax.experimental import pallas as pl
from jax.experimental.pallas import tpu as pltpu
from jax.experimental.pallas import tpu_sc as plsc

N_NODES = 10000
E_EDGES = 320000
FEAT = 128
NEG_SLOPE = 0.2

NC = 2
NS = 16
NTILES = NC * NS
CHUNK = E_EDGES // NTILES
LANES = 16
N_PAD = 10240
ROWS = N_PAD // FEAT


def _project_body(x_ref, w_ref, asr_ref, ads_ref, fcw_ref, out_ref):
    b = jnp.concatenate([asr_ref[...], ads_ref[...], fcw_ref[...]], axis=0)
    g = lax.dot_general(b, w_ref[...], (((1,), (1,)), ((), ())),
                        preferred_element_type=jnp.float32)
    p = lax.dot_general(g, x_ref[...], (((1,), (1,)), ((), ())),
                        preferred_element_type=jnp.float32)
    out_ref[...] = p.reshape(3, 8, FEAT)


def _project(x, W, asr, ads, fcw):
    grid = 10
    bn = N_PAD // grid
    return pl.pallas_call(
        _project_body,
        grid=(grid,),
        in_specs=[
            pl.BlockSpec((bn, FEAT), lambda i: (i, 0)),
            pl.BlockSpec((FEAT, FEAT), lambda i: (0, 0)),
            pl.BlockSpec((1, FEAT), lambda i: (0, 0)),
            pl.BlockSpec((1, FEAT), lambda i: (0, 0)),
            pl.BlockSpec((1, FEAT), lambda i: (0, 0)),
        ],
        out_specs=pl.BlockSpec((3, bn // FEAT, FEAT), lambda i: (0, i, 0)),
        out_shape=jax.ShapeDtypeStruct((3, ROWS, FEAT), jnp.float32),
    )(x, W, asr, ads, fcw)


def _edge_body(pt_hbm, src_hbm, dst_hbm, ew_hbm, parts_hbm,
               asrc_v, adst_v, z_v, srcs_v, dsts_v, ews_v, accn_v, accd_v):
    c = lax.axis_index("c")
    s = lax.axis_index("s")
    tid = s * NC + c
    base = tid * CHUNK

    pltpu.sync_copy(pt_hbm.at[0], asrc_v)
    pltpu.sync_copy(pt_hbm.at[1], adst_v)
    pltpu.sync_copy(pt_hbm.at[2], z_v)
    pltpu.sync_copy(src_hbm.at[pl.ds(base, CHUNK)], srcs_v)
    pltpu.sync_copy(dst_hbm.at[pl.ds(base, CHUNK)], dsts_v)
    pltpu.sync_copy(ew_hbm.at[pl.ds(base, CHUNK)], ews_v)

    zeros = jnp.zeros((LANES,), jnp.float32)

    @plsc.parallel_loop(0, N_PAD, step=LANES, unroll=8)
    def _zero(i):
        r = lax.shift_right_logical(i, 7)
        col = jnp.bitwise_and(i, FEAT - 1)
        accn_v[r, pl.ds(col, LANES)] = zeros
        accd_v[r, pl.ds(col, LANES)] = zeros

    @plsc.parallel_loop(0, CHUNK, step=LANES, unroll=5)
    def _edges(i):
        sl = pl.ds(i, LANES)
        sidx = srcs_v[sl]
        didx = dsts_v[sl]
        ew = ews_v[sl]
        sr = lax.shift_right_logical(sidx, 7)
        sc_ = jnp.bitwise_and(sidx, FEAT - 1)
        dr = lax.shift_right_logical(didx, 7)
        dc = jnp.bitwise_and(didx, FEAT - 1)
        ga = plsc.load_gather(asrc_v, [sr, sc_])
        gb = plsc.load_gather(adst_v, [dr, dc])
        gz = plsc.load_gather(z_v, [sr, sc_])
        a = ga + gb
        a = jnp.where(a >= 0.0, a, a * NEG_SLOPE)
        ex = jnp.exp(a)
        plsc.addupdate_scatter(accd_v, [dr, dc], ex)
        plsc.addupdate_scatter(accn_v, [dr, dc], ex * ew * gz)

    pltpu.sync_copy(accn_v, parts_hbm.at[tid, 0])
    pltpu.sync_copy(accd_v, parts_hbm.at[tid, 1])


def _edge_phase(pt, src, dst, ew):
    fn = pl.kernel(
        _edge_body,
        out_type=jax.ShapeDtypeStruct((NTILES, 2, ROWS, FEAT), jnp.float32),
        mesh=plsc.VectorSubcoreMesh(core_axis_name="c", subcore_axis_name="s",
                                    num_cores=NC, num_subcores=NS),
        scratch_types=[
            pltpu.VMEM((ROWS, FEAT), jnp.float32),
            pltpu.VMEM((ROWS, FEAT), jnp.float32),
            pltpu.VMEM((ROWS, FEAT), jnp.float32),
            pltpu.VMEM((CHUNK,), jnp.int32),
            pltpu.VMEM((CHUNK,), jnp.int32),
            pltpu.VMEM((CHUNK,), jnp.float32),
            pltpu.VMEM((ROWS, FEAT), jnp.float32),
            pltpu.VMEM((ROWS, FEAT), jnp.float32),
        ],
        compiler_params=pltpu.CompilerParams(needs_layout_passes=False),
    )
    return fn(pt, src, dst, ew)


def _finish_body(parts_ref, bias_ref, fcw_ref, fcb_ref, out_ref):
    num = jnp.sum(parts_ref[:, 0, :, :], axis=0)
    den = jnp.sum(parts_ref[:, 1, :, :], axis=0)
    const = jnp.sum(bias_ref[...] * fcw_ref[...]) + fcb_ref[0, 0]
    out_ref[...] = num / (den + 1e-16) + const


def _finish(parts, bias, fcw, fc_b):
    return pl.pallas_call(
        _finish_body,
        out_shape=jax.ShapeDtypeStruct((ROWS, FEAT), jnp.float32),
    )(parts, bias.reshape(1, FEAT), fcw, fc_b.reshape(1, 1))


@jax.jit
def kernel(x, edge_index, edge_weight, W, att_src, att_dst, bias, fc_W, fc_b):
    fcw = fc_W.reshape(1, FEAT)
    pt = _project(x, W, att_src, att_dst, fcw)
    parts = _edge_phase(pt, edge_index[0], edge_index[1], edge_weight)
    out = _finish(parts, bias, fcw, fc_b)
    return out.reshape(-1)[:N_NODES].reshape(N_NODES, 1)


# speedup vs baseline: 256.9851x; 1.3598x over previous
import jax
import jax.numpy as jnp
from jax import lax
from jax.experimental import pallas as pl
from jax.experimental.pallas import tpu as pltpu
from jax.experimental.pallas import tpu_sc as plsc

N_NODES = 10000
E_EDGES = 320000
FEAT = 128
NEG_SLOPE = 0.2

NC = 2
NS = 16
NTILES = NC * NS
CHUNK = E_EDGES // NTILES
WIN = 10240
LANES = 16
N_PAD = 10240
ROWS = N_PAD // FEAT


def _project_body(x_ref, w_ref, asr_ref, ads_ref, fcw_ref, out_ref):
    b = jnp.concatenate([asr_ref[...], ads_ref[...], fcw_ref[...]], axis=0)
    g = lax.dot_general(b, w_ref[...], (((1,), (1,)), ((), ())),
                        preferred_element_type=jnp.float32)
    p = lax.dot_general(g, x_ref[...], (((1,), (1,)), ((), ())),
                        preferred_element_type=jnp.float32)
    out_ref[...] = p.reshape(3, 8, FEAT)


def _project(x, W, asr, ads, fcw):
    grid = 10
    bn = N_PAD // grid
    return pl.pallas_call(
        _project_body,
        grid=(grid,),
        in_specs=[
            pl.BlockSpec((bn, FEAT), lambda i: (i, 0)),
            pl.BlockSpec((FEAT, FEAT), lambda i: (0, 0)),
            pl.BlockSpec((1, FEAT), lambda i: (0, 0)),
            pl.BlockSpec((1, FEAT), lambda i: (0, 0)),
            pl.BlockSpec((1, FEAT), lambda i: (0, 0)),
        ],
        out_specs=pl.BlockSpec((3, bn // FEAT, FEAT), lambda i: (0, i, 0)),
        out_shape=jax.ShapeDtypeStruct((3, ROWS, FEAT), jnp.float32),
    )(x, W, asr, ads, fcw)


def _edge_body(pt_hbm, ei_hbm, ew_hbm, parts_hbm,
               asrc_v, adst_v, z_v, sd_v, ews_v, accn_v, accd_v):
    c = lax.axis_index("c")
    s = lax.axis_index("s")
    tid = s * NC + c
    base = tid * CHUNK
    abase = jnp.minimum((base // 128) * 128, E_EDGES - WIN)
    off = base - abase

    pltpu.sync_copy(pt_hbm.at[0], asrc_v)
    pltpu.sync_copy(pt_hbm.at[1], adst_v)
    pltpu.sync_copy(pt_hbm.at[2], z_v)
    pltpu.sync_copy(ei_hbm.at[:, pl.ds(abase, WIN)], sd_v)
    pltpu.sync_copy(ew_hbm.at[pl.ds(base, CHUNK)], ews_v)

    zeros = jnp.zeros((LANES,), jnp.float32)

    @plsc.parallel_loop(0, N_PAD, step=LANES, unroll=8)
    def _zero(i):
        r = lax.shift_right_logical(i, 7)
        col = jnp.bitwise_and(i, FEAT - 1)
        accn_v[r, pl.ds(col, LANES)] = zeros
        accd_v[r, pl.ds(col, LANES)] = zeros

    @plsc.parallel_loop(0, CHUNK, step=LANES, unroll=5)
    def _edges(i):
        sl = pl.ds(i, LANES)
        el = pl.ds(off + i, LANES)
        sidx = sd_v[0, el]
        didx = sd_v[1, el]
        ew = ews_v[sl]
        sr = lax.shift_right_logical(sidx, 7)
        sc_ = jnp.bitwise_and(sidx, FEAT - 1)
        dr = lax.shift_right_logical(didx, 7)
        dc = jnp.bitwise_and(didx, FEAT - 1)
        ga = plsc.load_gather(asrc_v, [sr, sc_])
        gb = plsc.load_gather(adst_v, [dr, dc])
        gz = plsc.load_gather(z_v, [sr, sc_])
        a = ga + gb
        a = jnp.where(a >= 0.0, a, a * NEG_SLOPE)
        ex = jnp.exp(a)
        plsc.addupdate_scatter(accd_v, [dr, dc], ex)
        plsc.addupdate_scatter(accn_v, [dr, dc], ex * ew * gz)

    pltpu.sync_copy(accn_v, parts_hbm.at[tid, 0])
    pltpu.sync_copy(accd_v, parts_hbm.at[tid, 1])


def _edge_phase(pt, ei, ew):
    fn = pl.kernel(
        _edge_body,
        out_type=jax.ShapeDtypeStruct((NTILES, 2, ROWS, FEAT), jnp.float32),
        mesh=plsc.VectorSubcoreMesh(core_axis_name="c", subcore_axis_name="s",
                                    num_cores=NC, num_subcores=NS),
        scratch_types=[
            pltpu.VMEM((ROWS, FEAT), jnp.float32),
            pltpu.VMEM((ROWS, FEAT), jnp.float32),
            pltpu.VMEM((ROWS, FEAT), jnp.float32),
            pltpu.VMEM((2, WIN), jnp.int32),
            pltpu.VMEM((CHUNK,), jnp.float32),
            pltpu.VMEM((ROWS, FEAT), jnp.float32),
            pltpu.VMEM((ROWS, FEAT), jnp.float32),
        ],
        compiler_params=pltpu.CompilerParams(needs_layout_passes=False),
    )
    return fn(pt, ei, ew)


def _finish_body(parts_ref, bias_ref, fcw_ref, fcb_ref, out_ref):
    num = jnp.sum(parts_ref[:, 0, :, :], axis=0)
    den = jnp.sum(parts_ref[:, 1, :, :], axis=0)
    const = jnp.sum(bias_ref[...] * fcw_ref[...]) + fcb_ref[0, 0]
    out_ref[...] = num / (den + 1e-16) + const


def _finish(parts, bias, fcw, fc_b):
    return pl.pallas_call(
        _finish_body,
        out_shape=jax.ShapeDtypeStruct((ROWS, FEAT), jnp.float32),
    )(parts, bias.reshape(1, FEAT), fcw, fc_b.reshape(1, 1))


@jax.jit
def kernel(x, edge_index, edge_weight, W, att_src, att_dst, bias, fc_W, fc_b):
    fcw = fc_W.reshape(1, FEAT)
    pt = _project(x, W, att_src, att_dst, fcw)
    parts = _edge_phase(pt, edge_index, edge_weight)
    out = _finish(parts, bias, fcw, fc_b)
    return out.reshape(-1)[:N_NODES].reshape(N_NODES, 1)


# speedup vs baseline: 282.0021x; 1.0973x over previous
import jax
import jax.numpy as jnp
from jax import lax
from jax.experimental import pallas as pl
from jax.experimental.pallas import tpu as pltpu
from jax.experimental.pallas import tpu_sc as plsc

N_NODES = 10000
E_EDGES = 320000
FEAT = 128
NEG_SLOPE = 0.2

NC = 2
NS = 16
NTILES = NC * NS
CHUNK = E_EDGES // NTILES
WIN = 10240
LANES = 16
N_PAD = 10240
ROWS = N_PAD // FEAT


def _project_body(x_ref, w_ref, asr_ref, ads_ref, fcw_ref, out_ref):
    b = jnp.concatenate([asr_ref[...], ads_ref[...], fcw_ref[...]], axis=0)
    g = lax.dot_general(b, w_ref[...], (((1,), (1,)), ((), ())),
                        preferred_element_type=jnp.float32)
    p = lax.dot_general(g, x_ref[...], (((1,), (1,)), ((), ())),
                        preferred_element_type=jnp.float32)
    out_ref[...] = p.reshape(3, 8, FEAT)


def _project(x, W, asr, ads, fcw):
    grid = 10
    bn = N_PAD // grid
    return pl.pallas_call(
        _project_body,
        grid=(grid,),
        in_specs=[
            pl.BlockSpec((bn, FEAT), lambda i: (i, 0)),
            pl.BlockSpec((FEAT, FEAT), lambda i: (0, 0)),
            pl.BlockSpec((1, FEAT), lambda i: (0, 0)),
            pl.BlockSpec((1, FEAT), lambda i: (0, 0)),
            pl.BlockSpec((1, FEAT), lambda i: (0, 0)),
        ],
        out_specs=pl.BlockSpec((3, bn // FEAT, FEAT), lambda i: (0, i, 0)),
        out_shape=jax.ShapeDtypeStruct((3, ROWS, FEAT), jnp.float32),
    )(x, W, asr, ads, fcw)


def _edge_body(pt_hbm, ei_hbm, ew_hbm, parts_hbm,
               asrc_v, adst_v, z_v, sd_v, ews_v, accn_v, accd_v, sem):
    c = lax.axis_index("c")
    s = lax.axis_index("s")
    tid = s * NC + c
    base = tid * CHUNK
    abase = jnp.minimum((base // 128) * 128, E_EDGES - WIN)
    off = base - abase

    h1 = pltpu.async_copy(pt_hbm.at[0], asrc_v, sem)
    h2 = pltpu.async_copy(pt_hbm.at[1], adst_v, sem)
    h3 = pltpu.async_copy(pt_hbm.at[2], z_v, sem)
    h4 = pltpu.async_copy(ei_hbm.at[:, pl.ds(abase, WIN)], sd_v, sem)
    h5 = pltpu.async_copy(ew_hbm.at[pl.ds(base, CHUNK)], ews_v, sem)

    zeros = jnp.zeros((LANES,), jnp.float32)

    @plsc.parallel_loop(0, N_PAD, step=LANES, unroll=8)
    def _zero(i):
        r = lax.shift_right_logical(i, 7)
        col = jnp.bitwise_and(i, FEAT - 1)
        accn_v[r, pl.ds(col, LANES)] = zeros
        accd_v[r, pl.ds(col, LANES)] = zeros

    h1.wait()
    h2.wait()
    h3.wait()
    h4.wait()
    h5.wait()

    @plsc.parallel_loop(0, CHUNK, step=LANES, unroll=5)
    def _edges(i):
        sl = pl.ds(i, LANES)
        el = pl.ds(off + i, LANES)
        sidx = sd_v[0, el]
        didx = sd_v[1, el]
        ew = ews_v[sl]
        sr = lax.shift_right_logical(sidx, 7)
        sc_ = jnp.bitwise_and(sidx, FEAT - 1)
        dr = lax.shift_right_logical(didx, 7)
        dc = jnp.bitwise_and(didx, FEAT - 1)
        ga = plsc.load_gather(asrc_v, [sr, sc_])
        gb = plsc.load_gather(adst_v, [dr, dc])
        gz = plsc.load_gather(z_v, [sr, sc_])
        a = ga + gb
        a = jnp.where(a >= 0.0, a, a * NEG_SLOPE)
        ex = jnp.exp(a)
        plsc.addupdate_scatter(accd_v, [dr, dc], ex)
        plsc.addupdate_scatter(accn_v, [dr, dc], ex * ew * gz)

    pltpu.sync_copy(accn_v, parts_hbm.at[tid, 0])
    pltpu.sync_copy(accd_v, parts_hbm.at[tid, 1])


def _edge_phase(pt, ei, ew):
    fn = pl.kernel(
        _edge_body,
        out_type=jax.ShapeDtypeStruct((NTILES, 2, ROWS, FEAT), jnp.float32),
        mesh=plsc.VectorSubcoreMesh(core_axis_name="c", subcore_axis_name="s",
                                    num_cores=NC, num_subcores=NS),
        scratch_types=[
            pltpu.VMEM((ROWS, FEAT), jnp.float32),
            pltpu.VMEM((ROWS, FEAT), jnp.float32),
            pltpu.VMEM((ROWS, FEAT), jnp.float32),
            pltpu.VMEM((2, WIN), jnp.int32),
            pltpu.VMEM((CHUNK,), jnp.float32),
            pltpu.VMEM((ROWS, FEAT), jnp.float32),
            pltpu.VMEM((ROWS, FEAT), jnp.float32),
            pltpu.SemaphoreType.DMA,
        ],
        compiler_params=pltpu.CompilerParams(needs_layout_passes=False),
    )
    return fn(pt, ei, ew)


def _finish_body(parts_ref, bias_ref, fcw_ref, fcb_ref, out_ref):
    num = jnp.sum(parts_ref[:, 0, :, :], axis=0)
    den = jnp.sum(parts_ref[:, 1, :, :], axis=0)
    const = jnp.sum(bias_ref[...] * fcw_ref[...]) + fcb_ref[0, 0]
    out_ref[...] = num / (den + 1e-16) + const


def _finish(parts, bias, fcw, fc_b):
    return pl.pallas_call(
        _finish_body,
        out_shape=jax.ShapeDtypeStruct((ROWS, FEAT), jnp.float32),
    )(parts, bias.reshape(1, FEAT), fcw, fc_b.reshape(1, 1))


@jax.jit
def kernel(x, edge_index, edge_weight, W, att_src, att_dst, bias, fc_W, fc_b):
    fcw = fc_W.reshape(1, FEAT)
    pt = _project(x, W, att_src, att_dst, fcw)
    parts = _edge_phase(pt, edge_index, edge_weight)
    out = _finish(parts, bias, fcw, fc_b)
    return out.reshape(-1)[:N_NODES].reshape(N_NODES, 1)


# speedup vs baseline: 301.5337x; 1.0693x over previous
import jax
import jax.numpy as jnp
from jax import lax
from jax.experimental import pallas as pl
from jax.experimental.pallas import tpu as pltpu
from jax.experimental.pallas import tpu_sc as plsc

N_NODES = 10000
E_EDGES = 320000
FEAT = 128
NEG_SLOPE = 0.2

NC = 2
NS = 16
NTILES = NC * NS
CHUNK = E_EDGES // NTILES
WIN = 10240
LANES = 16
N_PAD = 10240
ROWS = N_PAD // FEAT


def _project_body(x_ref, w_ref, asr_ref, ads_ref, fcw_ref, out_ref):
    b = jnp.concatenate([asr_ref[...], ads_ref[...], fcw_ref[...]], axis=0)
    g = lax.dot_general(b, w_ref[...], (((1,), (1,)), ((), ())),
                        preferred_element_type=jnp.float32)
    p = lax.dot_general(g, x_ref[...], (((1,), (1,)), ((), ())),
                        preferred_element_type=jnp.float32)
    out_ref[...] = p.reshape(3, -1, FEAT)


def _project(x, W, asr, ads, fcw):
    grid = 5
    bn = N_PAD // grid
    return pl.pallas_call(
        _project_body,
        grid=(grid,),
        in_specs=[
            pl.BlockSpec((bn, FEAT), lambda i: (i, 0)),
            pl.BlockSpec((FEAT, FEAT), lambda i: (0, 0)),
            pl.BlockSpec((1, FEAT), lambda i: (0, 0)),
            pl.BlockSpec((1, FEAT), lambda i: (0, 0)),
            pl.BlockSpec((1, FEAT), lambda i: (0, 0)),
        ],
        out_specs=pl.BlockSpec((3, bn // FEAT, FEAT), lambda i: (0, i, 0)),
        out_shape=jax.ShapeDtypeStruct((3, ROWS, FEAT), jnp.float32),
    )(x, W, asr, ads, fcw)


def _edge_body(pt_hbm, ei_hbm, ew_hbm, parts_hbm,
               asrc_v, adst_v, z_v, sd_v, ews_v, accn_v, accd_v, sem):
    c = lax.axis_index("c")
    s = lax.axis_index("s")
    tid = s * NC + c
    base = tid * CHUNK
    abase = jnp.minimum((base // 128) * 128, E_EDGES - WIN)
    off = base - abase

    h1 = pltpu.async_copy(pt_hbm.at[0], asrc_v, sem)
    h2 = pltpu.async_copy(pt_hbm.at[1], adst_v, sem)
    h3 = pltpu.async_copy(pt_hbm.at[2], z_v, sem)
    h4 = pltpu.async_copy(ei_hbm.at[:, pl.ds(abase, WIN)], sd_v, sem)
    h5 = pltpu.async_copy(ew_hbm.at[pl.ds(base, CHUNK)], ews_v, sem)

    zeros = jnp.zeros((LANES,), jnp.float32)

    @plsc.parallel_loop(0, N_PAD, step=LANES, unroll=8)
    def _zero(i):
        r = lax.shift_right_logical(i, 7)
        col = jnp.bitwise_and(i, FEAT - 1)
        accn_v[r, pl.ds(col, LANES)] = zeros
        accd_v[r, pl.ds(col, LANES)] = zeros

    h1.wait()
    h2.wait()
    h3.wait()
    h4.wait()
    h5.wait()

    @plsc.parallel_loop(0, CHUNK, step=LANES, unroll=5)
    def _edges(i):
        sl = pl.ds(i, LANES)
        el = pl.ds(off + i, LANES)
        sidx = sd_v[0, el]
        didx = sd_v[1, el]
        ew = ews_v[sl]
        sr = lax.shift_right_logical(sidx, 7)
        sc_ = jnp.bitwise_and(sidx, FEAT - 1)
        dr = lax.shift_right_logical(didx, 7)
        dc = jnp.bitwise_and(didx, FEAT - 1)
        ga = plsc.load_gather(asrc_v, [sr, sc_])
        gb = plsc.load_gather(adst_v, [dr, dc])
        gz = plsc.load_gather(z_v, [sr, sc_])
        a = ga + gb
        a = jnp.where(a >= 0.0, a, a * NEG_SLOPE)
        ex = jnp.exp(a)
        plsc.addupdate_scatter(accd_v, [dr, dc], ex)
        plsc.addupdate_scatter(accn_v, [dr, dc], ex * ew * gz)

    pltpu.sync_copy(accn_v, parts_hbm.at[tid, 0])
    pltpu.sync_copy(accd_v, parts_hbm.at[tid, 1])


def _edge_phase(pt, ei, ew):
    fn = pl.kernel(
        _edge_body,
        out_type=jax.ShapeDtypeStruct((NTILES, 2, ROWS, FEAT), jnp.float32),
        mesh=plsc.VectorSubcoreMesh(core_axis_name="c", subcore_axis_name="s",
                                    num_cores=NC, num_subcores=NS),
        scratch_types=[
            pltpu.VMEM((ROWS, FEAT), jnp.float32),
            pltpu.VMEM((ROWS, FEAT), jnp.float32),
            pltpu.VMEM((ROWS, FEAT), jnp.float32),
            pltpu.VMEM((2, WIN), jnp.int32),
            pltpu.VMEM((CHUNK,), jnp.float32),
            pltpu.VMEM((ROWS, FEAT), jnp.float32),
            pltpu.VMEM((ROWS, FEAT), jnp.float32),
            pltpu.SemaphoreType.DMA,
        ],
        compiler_params=pltpu.CompilerParams(needs_layout_passes=False),
    )
    return fn(pt, ei, ew)


def _finish_body(parts_ref, bias_ref, fcw_ref, fcb_ref, out_ref):
    num = jnp.sum(parts_ref[:, 0, :, :], axis=0)
    den = jnp.sum(parts_ref[:, 1, :, :], axis=0)
    const = jnp.sum(bias_ref[...] * fcw_ref[...]) + fcb_ref[0, 0]
    out_ref[...] = num / (den + 1e-16) + const


def _finish(parts, bias, fcw, fc_b):
    return pl.pallas_call(
        _finish_body,
        out_shape=jax.ShapeDtypeStruct((ROWS, FEAT), jnp.float32),
    )(parts, bias.reshape(1, FEAT), fcw, fc_b.reshape(1, 1))


@jax.jit
def kernel(x, edge_index, edge_weight, W, att_src, att_dst, bias, fc_W, fc_b):
    fcw = fc_W.reshape(1, FEAT)
    pt = _project(x, W, att_src, att_dst, fcw)
    parts = _edge_phase(pt, edge_index, edge_weight)
    out = _finish(parts, bias, fcw, fc_b)
    return out.reshape(-1)[:N_NODES].reshape(N_NODES, 1)
